# baseline (speedup 1.0000x reference)
import functools

import jax
import jax.numpy as jnp
from jax import lax
from jax.experimental import pallas as pl
from jax.experimental.pallas import tpu as pltpu
from jax.experimental.pallas import tpu_sc as plsc

B = 2048
NPM = 40
N = B * NPM
N8 = N * 8
E = N * 4
NS = 16
RT = N // NS
EQ = E // 128
QT = EQ // NS
QC = 8
NCH = QT // QC


@functools.cache
def _sc_mesh():
    return plsc.VectorSubcoreMesh(
        core_axis_name="c", subcore_axis_name="s", num_cores=2, num_subcores=NS)


def _sc_deg(dstq, ones, zeros):
    @functools.partial(
        pl.kernel,
        out_type=jax.ShapeDtypeStruct((2, N, 16), jnp.float32),
        mesh=_sc_mesh(),
        compiler_params=pltpu.CompilerParams(use_tc_tiling_on_sc=False),
        scratch_types=[
            pltpu.VMEM_SHARED((N, 16), jnp.float32),
            pltpu.VMEM((QC, 128), jnp.int32),
            pltpu.VMEM((QC, 128, 16), jnp.float32),
            pltpu.SemaphoreType.DMA,
        ],
    )
    def k(dstq_h, ones_h, zeros_h, out_h, acc, idx_d, ones_v, sem):
        side = lax.axis_index("c")
        t = lax.axis_index("s")
        pltpu.sync_copy(ones_h, ones_v)
        pltpu.sync_copy(zeros_h.at[pl.ds(t * RT, RT)], acc.at[pl.ds(t * RT, RT)])
        plsc.subcore_barrier()

        def chunk(j, _):
            qb = t * QT + j * QC
            pltpu.sync_copy(dstq_h.at[side, pl.ds(qb, QC)], idx_d)
            descs = [
                pltpu.async_copy(ones_v.at[q], acc.at[idx_d.at[q]], sem, add=True)
                for q in range(QC)
            ]
            for d in descs:
                d.wait()
            return 0

        lax.fori_loop(0, NCH, chunk, 0)
        plsc.subcore_barrier()
        pltpu.sync_copy(acc.at[pl.ds(t * RT, RT)], out_h.at[side, pl.ds(t * RT, RT)])

    return k(dstq, ones, zeros)


def _sc_agg(yw, srcqm, dstq, zeros, cb_total, width):
    mult = width // 16
    y3 = yw.reshape(2, N * mult, 16)

    @functools.partial(
        pl.kernel,
        out_type=jax.ShapeDtypeStruct((2, N, mult, 16), jnp.float32),
        mesh=_sc_mesh(),
        compiler_params=pltpu.CompilerParams(use_tc_tiling_on_sc=False),
        scratch_types=[
            pltpu.VMEM_SHARED((N, 16), jnp.float32),
            pltpu.VMEM((2, QC, 128), jnp.int32),
            pltpu.VMEM((2, QC, 128), jnp.int32),
            pltpu.VMEM((QC, 128, 16), jnp.float32),
            pltpu.VMEM((QC, 128, 16), jnp.float32),
            pltpu.SemaphoreType.DMA,
            pltpu.SemaphoreType.DMA,
            pltpu.SemaphoreType.DMA,
            pltpu.SemaphoreType.DMA,
            pltpu.SemaphoreType.DMA,
            pltpu.SemaphoreType.DMA,
        ],
    )
    def k(y_h, srcqm_h, dstq_h, zeros_h, out_h, acc, idx_s, idx_d,
          rows0, rows1, l0sem, l1sem, g0sem, g1sem, s0sem, s1sem):
        side = lax.axis_index("c")
        t = lax.axis_index("s")
        yside = y_h.at[side]
        rows_b = (rows0, rows1)
        lsem_b = (l0sem, l1sem)
        gsem_b = (g0sem, g1sem)
        ssem_b = (s0sem, s1sem)

        def wait_all(descs):
            for d in descs:
                d.wait()

        def fire_l(cb, j, b):
            qb = t * QT + lax.min(j, NCH - 1) * QC
            return [
                pltpu.async_copy(srcqm_h.at[side, cb, pl.ds(qb, QC)],
                                 idx_s.at[b], lsem_b[b]),
                pltpu.async_copy(dstq_h.at[side, pl.ds(qb, QC)],
                                 idx_d.at[b], lsem_b[b]),
            ]

        def fire_g(ld, b):
            wait_all(ld)
            return [
                pltpu.async_copy(yside.at[idx_s.at[b, q]],
                                 rows_b[b].at[q], gsem_b[b])
                for q in range(QC)
            ]

        def fire_s(gd, b):
            wait_all(gd)
            return [
                pltpu.async_copy(rows_b[b].at[q], acc.at[idx_d.at[b, q]],
                                 ssem_b[b], add=True)
                for q in range(QC)
            ]

        for cb in range(cb_total):
            g0 = fire_g(fire_l(cb, 0, 0), 0)
            g1 = fire_g(fire_l(cb, 1, 1), 1)
            pltpu.sync_copy(zeros_h.at[pl.ds(t * RT, RT)],
                            acc.at[pl.ds(t * RT, RT)])
            plsc.subcore_barrier()

            def pair(i, _):
                a = 2 * i
                s0 = fire_s(g0, 0)
                s1 = fire_s(g1, 1)
                wait_all(s0)
                fire_g(fire_l(cb, a + 2, 0), 0)
                wait_all(s1)
                fire_g(fire_l(cb, a + 3, 1), 1)
                return 0

            lax.fori_loop(0, NCH // 2, pair, 0)
            for b in range(2):
                wait_all([pltpu.make_async_copy(zeros_h.at[pl.ds(0, 128)],
                                                rows_b[b].at[q], gsem_b[b])
                          for q in range(QC)])
            plsc.subcore_barrier()
            pltpu.sync_copy(acc.at[pl.ds(t * RT, RT)],
                            out_h.at[side, pl.ds(t * RT, RT), cb])

    return k(y3, srcqm, dstq, zeros).reshape(2, N, width)


_R = 512
_NB = N // _R

_blk = lambda: pl.BlockSpec((1, _R, 128), lambda s, i: (s, i, 0))


def _tc_prep(x_pad, dis_b):
    def body(x_ref, d_ref, y_ref):
        y_ref[0] = x_ref[0] * d_ref[0]

    return pl.pallas_call(
        body,
        grid=(2, _NB),
        in_specs=[_blk(), _blk()],
        out_specs=_blk(),
        out_shape=jax.ShapeDtypeStruct((2, N, 128), jnp.float32),
    )(x_pad, dis_b)


def _tc_mm1(s0, y0, dis_b, W0p, b0):
    def body(s_ref, y0_ref, d_ref, w0_ref, b0_ref, y_ref):
        dis = d_ref[0]
        t = ((s_ref[0] + y0_ref[0]) * dis)[:, :80]
        x1 = jnp.maximum(
            jnp.dot(t, w0_ref[...], preferred_element_type=jnp.float32)
            + b0_ref[0], 0.0)
        y_ref[0] = x1 * jnp.concatenate([dis, dis], axis=1)

    return pl.pallas_call(
        body,
        grid=(2, _NB),
        in_specs=[
            _blk(), _blk(), _blk(),
            pl.BlockSpec((80, 256), lambda s, i: (0, 0)),
            pl.BlockSpec((1, 256), lambda s, i: (0, 0)),
        ],
        out_specs=pl.BlockSpec((1, _R, 256), lambda s, i: (s, i, 0)),
        out_shape=jax.ShapeDtypeStruct((2, N, 256), jnp.float32),
    )(s0, y0, dis_b, W0p, b0)


def _tc_act(s1, y1, dis_b, W1, b1):
    def body(s_ref, y1_ref, d_ref, w1_ref, b1_ref, y_ref):
        dis = d_ref[0]
        agg2 = (s_ref[0] + y1_ref[0]) * jnp.concatenate([dis, dis], axis=1)
        x2 = jnp.maximum(
            jnp.dot(agg2, w1_ref[...], preferred_element_type=jnp.float32)
            + b1_ref[0], 0.0)
        y_ref[0] = x2 * dis

    blk256 = lambda: pl.BlockSpec((1, _R, 256), lambda s, i: (s, i, 0))
    return pl.pallas_call(
        body,
        grid=(2, _NB),
        in_specs=[blk256(), blk256(), _blk(),
                  pl.BlockSpec((256, 128), lambda s, i: (0, 0)),
                  pl.BlockSpec((1, 128), lambda s, i: (0, 0))],
        out_specs=_blk(),
        out_shape=jax.ShapeDtypeStruct((2, N, 128), jnp.float32),
    )(s1, y1, dis_b, W1, b1)


_RM = 640
_NM = N // _RM


def _tc_head(s2, y2, dis_b, W2, b2):
    def body(s_ref, y2_ref, d_ref, w2_ref, b2_ref, r_ref):
        t = (s_ref[0] + y2_ref[0]) * d_ref[0]
        x3 = jnp.maximum(
            jnp.dot(t, w2_ref[...], preferred_element_type=jnp.float32)
            + b2_ref[0], 0.0)
        r_ref[0] = jnp.max(x3.reshape(_RM // NPM, NPM, 400), axis=1)

    mblk = lambda: pl.BlockSpec((1, _RM, 128), lambda s, i: (s, i, 0))
    return pl.pallas_call(
        body,
        grid=(2, _NM),
        in_specs=[
            mblk(), mblk(), mblk(),
            pl.BlockSpec((128, 400), lambda s, i: (0, 0)),
            pl.BlockSpec((1, 400), lambda s, i: (0, 0)),
        ],
        out_specs=pl.BlockSpec((1, _RM // NPM, 400), lambda s, i: (s, i, 0)),
        out_shape=jax.ShapeDtypeStruct((2, B, 400), jnp.float32),
    )(s2, y2, dis_b, W2, b2)


_RB = 256


def _tc_final(ctx, feat, cW0, cb0, cW1, cb1, cW2, cb2,
              dW0, db0, dW1, db1, fA, fB, fC, fb0, fW1, fb1, fW2, fb2):
    def body(c_ref, f_ref, cw0, cb0r, cw1, cb1r, cw2, cb2r,
             dw0, db0r, dw1, db1r, fa, fb, fcr, fb0r, fw1, fb1r, fw2, fb2r,
             o_ref):
        c = c_ref[...]
        nrm = jnp.sqrt(jnp.sum(c * c, axis=1, keepdims=True))
        c = c / jnp.maximum(nrm, 1e-12)
        m = jnp.maximum(jnp.dot(c, cw0[...], preferred_element_type=jnp.float32) + cb0r[0], 0.0)
        m = jnp.maximum(jnp.dot(m, cw1[...], preferred_element_type=jnp.float32) + cb1r[0], 0.0)
        m = jnp.dot(m, cw2[...], preferred_element_type=jnp.float32) + cb2r[0]

        def dmlp(x):
            h = jnp.maximum(jnp.dot(x, dw0[...], preferred_element_type=jnp.float32) + db0r[0], 0.0)
            return jnp.dot(h, dw1[...], preferred_element_type=jnp.float32) + db1r[0]

        dl = dmlp(f_ref[0])
        dr = dmlp(f_ref[1])
        z = jnp.maximum(
            jnp.dot(m, fa[...], preferred_element_type=jnp.float32)
            + jnp.dot(dl, fb[...], preferred_element_type=jnp.float32)
            + jnp.dot(dr, fcr[...], preferred_element_type=jnp.float32)
            + fb0r[0], 0.0)
        z = jnp.maximum(jnp.dot(z, fw1[...], preferred_element_type=jnp.float32) + fb1r[0], 0.0)
        o_ref[...] = jnp.dot(z, fw2[...], preferred_element_type=jnp.float32) + fb2r[0]

    full = lambda a, b: pl.BlockSpec((a, b), lambda i: (0, 0))
    return pl.pallas_call(
        body,
        grid=(B // _RB,),
        in_specs=[
            pl.BlockSpec((_RB, 288), lambda i: (i, 0)),
            pl.BlockSpec((2, _RB, 400), lambda i: (0, i, 0)),
            full(288, 512), full(1, 512), full(512, 256), full(1, 256),
            full(256, 128), full(1, 128),
            full(400, 138), full(1, 138), full(138, 128), full(1, 128),
            full(128, 32), full(128, 32), full(128, 32), full(1, 32),
            full(32, 32), full(1, 32), full(32, 1), full(1, 1),
        ],
        out_specs=pl.BlockSpec((_RB, 1), lambda i: (i, 0)),
        out_shape=jax.ShapeDtypeStruct((B, 1), jnp.float32),
    )(ctx, feat, cW0, cb0, cW1, cb1, cW2, cb2,
      dW0, db0, dW1, db1, fA, fB, fC, fb0, fW1, fb1, fW2, fb2)


def kernel(x_left, x_right, edge_index_left, edge_index_right,
           graph_ids_left, graph_ids_right, context_features,
           ctx_W0, ctx_b0, ctx_W1, ctx_b1, ctx_W2, ctx_b2,
           gcn_W0, gcn_b0, gcn_W1, gcn_b1, gcn_W2, gcn_b2,
           dmlp_W0, dmlp_b0, dmlp_W1, dmlp_b1,
           fc_W0, fc_b0, fc_W1, fc_b1, fc_W2, fc_b2):
    f32 = jnp.float32
    src2 = jnp.stack([edge_index_left[0], edge_index_right[0]])
    dstq = jnp.stack([edge_index_left[1], edge_index_right[1]]).reshape(2, EQ, 128)
    srcq8 = (src2[:, None, :] * 8
             + jnp.arange(8, dtype=jnp.int32)[None, :, None]).reshape(2, 8, EQ, 128)
    srcq16 = (src2[:, None, :] * 16
              + jnp.arange(16, dtype=jnp.int32)[None, :, None]).reshape(2, 16, EQ, 128)
    x_pad = jnp.stack([
        jnp.pad(x_left, ((0, 0), (0, 59))),
        jnp.pad(x_right, ((0, 0), (0, 59))),
    ])
    W0p = jnp.pad(gcn_W0, ((0, 11), (0, 0)))
    ones = jnp.ones((QC, 128, 16), f32)
    zeros = jnp.zeros((N, 16), f32)

    deg_b = _sc_deg(dstq, ones, zeros)
    dis_b = jnp.broadcast_to(
        (1.0 / jnp.sqrt(deg_b[:, :, 0] + 1.0))[:, :, None], (2, N, 128))

    y0 = _tc_prep(x_pad, dis_b)
    s0 = _sc_agg(y0, srcq8, dstq, zeros, 5, 128)
    y1 = _tc_mm1(s0, y0, dis_b, W0p, gcn_b0.reshape(1, 256))
    s1 = _sc_agg(y1, srcq16, dstq, zeros, 16, 256)
    y2 = _tc_act(s1, y1, dis_b, gcn_W1, gcn_b1.reshape(1, 128))
    s2 = _sc_agg(y2, srcq8, dstq, zeros, 8, 128)
    feat = _tc_head(s2, y2, dis_b, gcn_W2, gcn_b2.reshape(1, 400))

    out = _tc_final(
        context_features, feat,
        ctx_W0, ctx_b0.reshape(1, 512), ctx_W1, ctx_b1.reshape(1, 256),
        ctx_W2, ctx_b2.reshape(1, 128),
        dmlp_W0, dmlp_b0.reshape(1, 138), dmlp_W1, dmlp_b1.reshape(1, 128),
        fc_W0[:128], fc_W0[128:256], fc_W0[256:384], fc_b0.reshape(1, 32),
        fc_W1, fc_b1.reshape(1, 32), fc_W2, fc_b2.reshape(1, 1),
    )
    return jnp.squeeze(out, axis=-1)
